# Initial kernel scaffold; baseline (speedup 1.0000x reference)
import jax
import jax.numpy as jnp
from jax.experimental import pallas as pl


def kernel(token_ids, tok_table, pos_table):
    raise NotImplementedError("write your pallas kernel here")


# speedup vs baseline: 1.0390x; 1.0390x over previous
import functools
import math

import jax
import jax.numpy as jnp
from jax import lax
from jax.experimental import pallas as pl
from jax.experimental.pallas import tpu as pltpu
from jax.experimental.pallas import tpu_sc as plsc

VOCAB = 50257
D_MODEL = 768
BATCH = 4
SEQ = 2048

NC = 2
NS = 16
LANES = 16
NW = NC * NS
NTOK = BATCH * SEQ
TPW = NTOK // NW
CHUNK = 64
NCH = TPW // CHUNK
VECS_PER_ROW = D_MODEL // LANES
SCALE = math.sqrt(D_MODEL)

_mesh = plsc.VectorSubcoreMesh(core_axis_name="c", subcore_axis_name="s")


@functools.partial(
    pl.kernel,
    out_type=jax.ShapeDtypeStruct((NTOK, D_MODEL), jnp.float32),
    mesh=_mesh,
    scratch_types=[
        pltpu.VMEM((NCH, CHUNK), jnp.int32),
        pltpu.VMEM((CHUNK, D_MODEL), jnp.float32),
        pltpu.VMEM((CHUNK, D_MODEL), jnp.float32),
        pltpu.SemaphoreType.DMA,
    ],
)
def _emb_kernel(ids_hbm, tok_hbm, pos_hbm, out_hbm, idx_v, tok_v, pos_v, sem):
    wid = lax.axis_index("s") * NC + lax.axis_index("c")
    base = wid * TPW
    s0 = (wid % (SEQ // TPW)) * TPW

    pltpu.sync_copy(ids_hbm.at[wid], idx_v)

    for k in range(NCH):
        gather = pltpu.async_copy(tok_hbm.at[idx_v.at[k]], tok_v, sem)
        pltpu.sync_copy(pos_hbm.at[pl.ds(s0 + k * CHUNK, CHUNK)], pos_v)
        gather.wait()

        def row_body(r, _):
            for l in range(VECS_PER_ROW):
                sl = pl.ds(l * LANES, LANES)
                tok_v[r, sl] = tok_v[r, sl] * SCALE + pos_v[r, sl]
            return _

        lax.fori_loop(0, CHUNK, row_body, 0, unroll=False)
        pltpu.sync_copy(tok_v, out_hbm.at[pl.ds(base + k * CHUNK, CHUNK)])


def kernel(token_ids, tok_table, pos_table):
    ids = jnp.reshape(token_ids.astype(jnp.int32), (NW, NCH, CHUNK))
    out = _emb_kernel(ids, tok_table, pos_table)
    return jnp.reshape(out, (BATCH, SEQ, D_MODEL))
